# baseline (speedup 1.0000x reference)
import functools

import jax
import jax.numpy as jnp
from jax import lax
from jax.experimental import pallas as pl
from jax.experimental.pallas import tpu as pltpu
from jax.experimental.pallas import tpu_sc as plsc

_STEPS = 3
_B = 512
_D = 128
_NEG = -1e30

_RB = 160
_NBLK = 625
_NW = 32
_NBORD = (_NBLK + _NW - 1) // _NW
_NSLOT = 4


def _sc_attn_body(x_hbm, b_hbm, q_hbm, mp_hbm, sp_hbm, vp_hbm,
                  q_loc, xb0, xb1, bb_all, m_loc, s_loc, stage,
                  sem, semb, semx0, semx1):
    nc = 2
    wid = lax.axis_index("s") * nc + lax.axis_index("c")

    pltpu.sync_copy(q_hbm, q_loc)

    neg16 = jnp.full((16,), _NEG, jnp.float32)
    zero16 = jnp.zeros((16,), jnp.float32)
    for i in range(_B // 16):
        m_loc[pl.ds(i * 16, 16)] = neg16
        s_loc[pl.ds(i * 16, 16)] = zero16

    def flush(cur_b, cur_m, cur_s, acc, fc):
        slot = lax.rem(fc, _NSLOT)

        @pl.when(fc >= _NSLOT)
        def _():
            pltpu.make_async_copy(
                vp_hbm.at[pl.ds(0, _D)],
                stage.at[pl.ds(slot * _D, _D)], sem).wait()

        for j in range(8):
            stage[pl.ds(slot * _D + 16 * j, 16)] = acc[j]
        pltpu.make_async_copy(
            stage.at[pl.ds(slot * _D, _D)],
            vp_hbm.at[pl.ds((wid * _B + cur_b) * _D, _D)], sem).start()
        base = (cur_b // 16) * 16
        lmask = lax.iota(jnp.int32, 16) == (cur_b - base)
        m_loc[pl.ds(base, 16)] = jnp.where(lmask, cur_m,
                                           m_loc[pl.ds(base, 16)])
        s_loc[pl.ds(base, 16)] = jnp.where(lmask, cur_s,
                                           s_loc[pl.ds(base, 16)])

    def make_row_body(xref, boff, vf16):
        def row_body(r, carry):
            cur_b, cur_m, cur_s, acc, fc = carry
            b = bb_all[pl.ds(boff + r, 16)][0]
            xv = []
            facc = zero16
            for j in range(8):
                xj = xref[pl.ds(r * _D + 16 * j, 16)]
                qj = q_loc[pl.ds(b * _D + 16 * j, 16)]
                xv.append(xj)
                facc = facc + xj * qj
            iota16 = lax.iota(jnp.int32, 16)
            e_v = facc
            for k in (1, 2, 4, 8):
                e_v = e_v + jnp.take(e_v, jnp.bitwise_xor(iota16, k))

            changed = b != cur_b
            do_flush = changed & (cur_b >= 0) & (cur_s[0] > 0.0)

            @pl.when(do_flush)
            def _():
                flush(cur_b, cur_m, cur_s, acc, fc)

            fc = jnp.where(do_flush, fc + 1, fc)
            keep = lax.broadcast_in_dim(
                jnp.where(changed, 0.0, 1.0), (16,), ())
            m_prev = cur_m * keep + neg16 * (1.0 - keep)
            s_prev = cur_s * keep
            m_new = jnp.maximum(m_prev, e_v)
            f_old = jnp.exp(m_prev - m_new)
            wgt = jnp.exp(e_v - m_new) * vf16
            s_new = s_prev * f_old + wgt
            acc_new = tuple(
                acc[j] * (keep * f_old) + wgt * xv[j] for j in range(8))
            return (b, m_new, s_new, acc_new, fc)
        return row_body

    def _blk(ib):
        blk_i = wid + ib * _NW
        return jnp.where(blk_i < _NBLK, blk_i, wid)

    def bpre(ib, z):
        pltpu.make_async_copy(b_hbm.at[pl.ds(_blk(ib) * _RB, _RB)],
                              bb_all.at[pl.ds(ib * _RB, _RB)], semb).start()
        return z
    lax.fori_loop(0, _NBORD, bpre, 0)

    def bdrain(ib, z):
        pltpu.make_async_copy(b_hbm.at[pl.ds(0, _RB)],
                              bb_all.at[pl.ds(0, _RB)], semb).wait()
        return z
    lax.fori_loop(0, _NBORD, bdrain, 0)

    def xdesc(ib, bref, bsem):
        return pltpu.make_async_copy(
            x_hbm.at[pl.ds(_blk(ib) * _RB * _D, _RB * _D)],
            bref.at[pl.ds(0, _RB * _D)], bsem)

    xdesc(0, xb0, semx0).start()
    xdesc(1, xb1, semx1).start()

    def blk_pair(i, carry):
        for par, bref, bsem in ((0, xb0, semx0), (1, xb1, semx1)):
            ib = i * 2 + par
            xdesc(ib, bref, bsem).wait()
            valid = (wid + ib * _NW) < _NBLK
            vf16 = lax.broadcast_in_dim(
                jnp.where(valid, 1.0, 0.0), (16,), ())
            carry = lax.fori_loop(0, _RB,
                                  make_row_body(bref, ib * _RB, vf16), carry)

            @pl.when(ib + 2 < _NBORD)
            def _(ib=ib, bref=bref, bsem=bsem):
                xdesc(ib + 2, bref, bsem).start()
        return carry

    init = (jnp.int32(-1), neg16, zero16, (zero16,) * 8, jnp.int32(0))
    cur_b, cur_m, cur_s, acc, fc = lax.fori_loop(
        0, _NBORD // 2, blk_pair, init)

    last_real = (cur_b >= 0) & (cur_s[0] > 0.0)

    @pl.when(last_real)
    def _():
        flush(cur_b, cur_m, cur_s, acc, fc)
    fc = jnp.where(last_real, fc + 1, fc)

    def drain(i, _):
        pltpu.make_async_copy(
            vp_hbm.at[pl.ds(0, _D)], stage.at[pl.ds(0, _D)], sem).wait()
        return 0
    lax.fori_loop(0, jnp.minimum(fc, _NSLOT), drain, 0)

    pltpu.sync_copy(m_loc, mp_hbm.at[pl.ds(wid * _B, _B)])
    pltpu.sync_copy(s_loc, sp_hbm.at[pl.ds(wid * _B, _B)])


def _fused_body(q_ref, c_ref, mp_ref, sp_ref, vp_ref,
                wih_ref, whh_ref, bias_ref, qs_out, h_out, c_out):
    d = q_ref.shape[1]
    r_iota = jax.lax.broadcasted_iota(jnp.int32, (_B, _B), 0)
    c_iota = jax.lax.broadcasted_iota(jnp.int32, (_B, _B), 1)
    ident = (r_iota == c_iota).astype(jnp.float32)
    mp_t = jax.lax.dot_general(ident, mp_ref[...], (((1,), (1,)), ((), ())),
                               preferred_element_type=jnp.float32)
    sp_t = jax.lax.dot_general(ident, sp_ref[...], (((1,), (1,)), ((), ())),
                               preferred_element_type=jnp.float32)
    alive = mp_t > (_NEG / 2)
    m = jnp.max(mp_t, axis=1, keepdims=True)
    coef = jnp.where(alive, jnp.exp(mp_t - m), 0.0)
    s = jnp.sum(coef * sp_t, axis=1, keepdims=True)
    v = jnp.zeros((_B, _D), jnp.float32)
    for w in range(_NW):
        cw = coef[:, w:w + 1]
        v = v + jnp.where(alive[:, w:w + 1], vp_ref[w] * cw, 0.0)
    r = v / (s + 1e-16)
    q_star = jnp.concatenate([q_ref[...], r], axis=1)
    qs_out[...] = q_star
    gates = (
        jax.lax.dot_general(q_star, wih_ref[...], (((1,), (1,)), ((), ())),
                            preferred_element_type=jnp.float32)
        + jax.lax.dot_general(q_ref[...], whh_ref[...],
                              (((1,), (1,)), ((), ())),
                              preferred_element_type=jnp.float32)
        + bias_ref[...]
    )
    i = jax.nn.sigmoid(gates[:, :d])
    f = jax.nn.sigmoid(gates[:, d:2 * d])
    g = jnp.tanh(gates[:, 2 * d:3 * d])
    o = jax.nn.sigmoid(gates[:, 3 * d:])
    c_new = f * c_ref[...] + i * g
    h_out[...] = o * jnp.tanh(c_new)
    c_out[...] = c_new


def kernel(x, batch, W_ih, W_hh, b_ih, b_hh):
    N, d = x.shape
    batch_i = batch.astype(jnp.int32)
    bias = (b_ih + b_hh).reshape(1, 4 * d).astype(jnp.float32)
    x1 = x.reshape(-1)

    fused_call = pl.pallas_call(
        _fused_body,
        out_shape=(jax.ShapeDtypeStruct((_B, 2 * d), jnp.float32),
                   jax.ShapeDtypeStruct((_B, d), jnp.float32),
                   jax.ShapeDtypeStruct((_B, d), jnp.float32)),
    )

    mesh = plsc.VectorSubcoreMesh(core_axis_name="c", subcore_axis_name="s")
    sc_attn = pl.kernel(
        _sc_attn_body,
        out_type=(jax.ShapeDtypeStruct((_NW * _B,), jnp.float32),
                  jax.ShapeDtypeStruct((_NW * _B,), jnp.float32),
                  jax.ShapeDtypeStruct((_NW * _B * _D,), jnp.float32)),
        mesh=mesh,
        scratch_types=[
            pltpu.VMEM((_B * _D,), jnp.float32),
            pltpu.VMEM((_RB * _D + _D,), jnp.float32),
            pltpu.VMEM((_RB * _D + _D,), jnp.float32),
            pltpu.VMEM((_NBORD * _RB + 16,), jnp.int32),
            pltpu.VMEM((_B,), jnp.float32),
            pltpu.VMEM((_B,), jnp.float32),
            pltpu.VMEM((_NSLOT * _D,), jnp.float32),
            pltpu.SemaphoreType.DMA,
            pltpu.SemaphoreType.DMA,
            pltpu.SemaphoreType.DMA,
            pltpu.SemaphoreType.DMA,
        ],
    )

    h = jnp.zeros((_B, d), jnp.float32)
    c = jnp.zeros((_B, d), jnp.float32)
    mp = jnp.full((_NW, _B), _NEG, jnp.float32)
    sp = jnp.zeros((_NW, _B), jnp.float32)
    vp = jnp.zeros((_NW, _B, d), jnp.float32)
    for _ in range(_STEPS):
        _, h, c = fused_call(h, c, mp, sp, vp, W_ih, W_hh, bias)
        mpf, spf, vpf = sc_attn(x1, batch_i, h.reshape(-1))
        mp = mpf.reshape(_NW, _B)
        sp = spf.reshape(_NW, _B)
        vp = vpf.reshape(_NW, _B, d)
    q_star, _, _ = fused_call(h, c, mp, sp, vp, W_ih, W_hh, bias)
    return q_star


# speedup vs baseline: 1.0387x; 1.0387x over previous
import functools

import jax
import jax.numpy as jnp
from jax import lax
from jax.experimental import pallas as pl
from jax.experimental.pallas import tpu as pltpu
from jax.experimental.pallas import tpu_sc as plsc

_STEPS = 3
_B = 512
_D = 128
_NEG = -1e30

_RB = 160
_NBLK = 625
_NW = 32
_NBORD = (_NBLK + _NW - 1) // _NW
_NSLOT = 4


def _sc_attn_body(x_hbm, b_hbm, q_hbm, mp_hbm, sp_hbm, vp_hbm,
                  q_loc, xb0, xb1, bb_all, m_loc, s_loc, stage,
                  sem, semb, semx0, semx1):
    nc = 2
    wid = lax.axis_index("s") * nc + lax.axis_index("c")

    pltpu.sync_copy(q_hbm, q_loc)

    neg16 = jnp.full((16,), _NEG, jnp.float32)
    zero16 = jnp.zeros((16,), jnp.float32)
    for i in range(_B // 16):
        m_loc[pl.ds(i * 16, 16)] = neg16
        s_loc[pl.ds(i * 16, 16)] = zero16

    def flush(cur_b, cur_m, cur_s, acc, fc):
        slot = lax.rem(fc, _NSLOT)

        @pl.when(fc >= _NSLOT)
        def _():
            pltpu.make_async_copy(
                vp_hbm.at[pl.ds(0, _D)],
                stage.at[pl.ds(slot * _D, _D)], sem).wait()

        for j in range(8):
            stage[pl.ds(slot * _D + 16 * j, 16)] = acc[j]
        pltpu.make_async_copy(
            stage.at[pl.ds(slot * _D, _D)],
            vp_hbm.at[pl.ds((wid * _B + cur_b) * _D, _D)], sem).start()
        base = (cur_b // 16) * 16
        lmask = lax.iota(jnp.int32, 16) == (cur_b - base)
        m_loc[pl.ds(base, 16)] = jnp.where(lmask, cur_m,
                                           m_loc[pl.ds(base, 16)])
        s_loc[pl.ds(base, 16)] = jnp.where(lmask, cur_s,
                                           s_loc[pl.ds(base, 16)])

    def make_row_body(xref, boff, vf16):
        def row_body(r, carry):
            cur_b, cur_m, cur_s, acc, fc = carry
            b = bb_all[pl.ds(boff + r, 16)][0]
            xv = []
            facc = zero16
            for j in range(8):
                xj = xref[pl.ds(r * _D + 16 * j, 16)]
                qj = q_loc[pl.ds(b * _D + 16 * j, 16)]
                xv.append(xj)
                facc = facc + xj * qj
            iota16 = lax.iota(jnp.int32, 16)
            e_v = facc
            for k in (1, 2, 4, 8):
                e_v = e_v + jnp.take(e_v, jnp.bitwise_xor(iota16, k))

            changed = b != cur_b
            do_flush = changed & (cur_b >= 0) & (cur_s[0] > 0.0)

            @pl.when(do_flush)
            def _():
                flush(cur_b, cur_m, cur_s, acc, fc)

            fc = jnp.where(do_flush, fc + 1, fc)
            keep = lax.broadcast_in_dim(
                jnp.where(changed, 0.0, 1.0), (16,), ())
            m_prev = cur_m * keep + neg16 * (1.0 - keep)
            s_prev = cur_s * keep
            m_new = jnp.maximum(m_prev, e_v)
            f_old = jnp.exp(m_prev - m_new)
            wgt = jnp.exp(e_v - m_new) * vf16
            s_new = s_prev * f_old + wgt
            acc_new = tuple(
                acc[j] * (keep * f_old) + wgt * xv[j] for j in range(8))
            return (b, m_new, s_new, acc_new, fc)
        return row_body

    def _blk(ib):
        blk_i = wid + ib * _NW
        return jnp.where(blk_i < _NBLK, blk_i, wid)

    def bpre(ib, z):
        pltpu.make_async_copy(b_hbm.at[pl.ds(_blk(ib) * _RB, _RB)],
                              bb_all.at[pl.ds(ib * _RB, _RB)], semb).start()
        return z
    lax.fori_loop(0, _NBORD, bpre, 0)

    def bdrain(ib, z):
        pltpu.make_async_copy(b_hbm.at[pl.ds(0, _RB)],
                              bb_all.at[pl.ds(0, _RB)], semb).wait()
        return z
    lax.fori_loop(0, _NBORD, bdrain, 0)

    def xdesc(ib, bref, bsem):
        return pltpu.make_async_copy(
            x_hbm.at[pl.ds(_blk(ib) * _RB * _D, _RB * _D)],
            bref.at[pl.ds(0, _RB * _D)], bsem)

    xdesc(0, xb0, semx0).start()
    xdesc(1, xb1, semx1).start()

    def blk_pair(i, carry):
        for par, bref, bsem in ((0, xb0, semx0), (1, xb1, semx1)):
            ib = i * 2 + par
            xdesc(ib, bref, bsem).wait()
            valid = (wid + ib * _NW) < _NBLK
            vf16 = lax.broadcast_in_dim(
                jnp.where(valid, 1.0, 0.0), (16,), ())
            carry = lax.fori_loop(0, _RB,
                                  make_row_body(bref, ib * _RB, vf16), carry)

            @pl.when(ib + 2 < _NBORD)
            def _(ib=ib, bref=bref, bsem=bsem):
                xdesc(ib + 2, bref, bsem).start()
        return carry

    init = (jnp.int32(-1), neg16, zero16, (zero16,) * 8, jnp.int32(0))
    cur_b, cur_m, cur_s, acc, fc = lax.fori_loop(
        0, _NBORD // 2, blk_pair, init)

    last_real = (cur_b >= 0) & (cur_s[0] > 0.0)

    @pl.when(last_real)
    def _():
        flush(cur_b, cur_m, cur_s, acc, fc)
    fc = jnp.where(last_real, fc + 1, fc)

    def drain(i, _):
        pltpu.make_async_copy(
            vp_hbm.at[pl.ds(0, _D)], stage.at[pl.ds(0, _D)], sem).wait()
        return 0
    lax.fori_loop(0, jnp.minimum(fc, _NSLOT), drain, 0)

    pltpu.sync_copy(m_loc, mp_hbm.at[pl.ds(wid * _B, _B)])
    pltpu.sync_copy(s_loc, sp_hbm.at[pl.ds(wid * _B, _B)])


def _lstm0_body(bias_ref, h_out, c_out):
    d = h_out.shape[1]
    gates = jnp.broadcast_to(bias_ref[...], (_B, 4 * d))
    i = jax.nn.sigmoid(gates[:, :d])
    g = jnp.tanh(gates[:, 2 * d:3 * d])
    c_new = i * g
    h_out[...] = jax.nn.sigmoid(gates[:, 3 * d:]) * jnp.tanh(c_new)
    c_out[...] = c_new


def _fused_body(q_ref, c_ref, mp_ref, sp_ref, vp_ref,
                wih_ref, whh_ref, bias_ref, qs_out, h_out, c_out):
    d = q_ref.shape[1]
    r_iota = jax.lax.broadcasted_iota(jnp.int32, (_B, _B), 0)
    c_iota = jax.lax.broadcasted_iota(jnp.int32, (_B, _B), 1)
    ident = (r_iota == c_iota).astype(jnp.float32)
    mp_t = jax.lax.dot_general(ident, mp_ref[...], (((1,), (1,)), ((), ())),
                               preferred_element_type=jnp.float32)
    sp_t = jax.lax.dot_general(ident, sp_ref[...], (((1,), (1,)), ((), ())),
                               preferred_element_type=jnp.float32)
    alive = mp_t > (_NEG / 2)
    m = jnp.max(mp_t, axis=1, keepdims=True)
    coef = jnp.where(alive, jnp.exp(mp_t - m), 0.0)
    s = jnp.sum(coef * sp_t, axis=1, keepdims=True)
    v = jnp.zeros((_B, _D), jnp.float32)
    for w in range(_NW):
        cw = coef[:, w:w + 1]
        v = v + jnp.where(alive[:, w:w + 1], vp_ref[w] * cw, 0.0)
    r = v / (s + 1e-16)
    q_star = jnp.concatenate([q_ref[...], r], axis=1)
    qs_out[...] = q_star
    gates = (
        jax.lax.dot_general(q_star, wih_ref[...], (((1,), (1,)), ((), ())),
                            preferred_element_type=jnp.float32)
        + jax.lax.dot_general(q_ref[...], whh_ref[...],
                              (((1,), (1,)), ((), ())),
                              preferred_element_type=jnp.float32)
        + bias_ref[...]
    )
    i = jax.nn.sigmoid(gates[:, :d])
    f = jax.nn.sigmoid(gates[:, d:2 * d])
    g = jnp.tanh(gates[:, 2 * d:3 * d])
    o = jax.nn.sigmoid(gates[:, 3 * d:])
    c_new = f * c_ref[...] + i * g
    h_out[...] = o * jnp.tanh(c_new)
    c_out[...] = c_new


def kernel(x, batch, W_ih, W_hh, b_ih, b_hh):
    N, d = x.shape
    batch_i = batch.astype(jnp.int32)
    bias = (b_ih + b_hh).reshape(1, 4 * d).astype(jnp.float32)
    x1 = x.reshape(-1)

    fused_call = pl.pallas_call(
        _fused_body,
        out_shape=(jax.ShapeDtypeStruct((_B, 2 * d), jnp.float32),
                   jax.ShapeDtypeStruct((_B, d), jnp.float32),
                   jax.ShapeDtypeStruct((_B, d), jnp.float32)),
    )

    mesh = plsc.VectorSubcoreMesh(core_axis_name="c", subcore_axis_name="s")
    sc_attn = pl.kernel(
        _sc_attn_body,
        out_type=(jax.ShapeDtypeStruct((_NW * _B,), jnp.float32),
                  jax.ShapeDtypeStruct((_NW * _B,), jnp.float32),
                  jax.ShapeDtypeStruct((_NW * _B * _D,), jnp.float32)),
        mesh=mesh,
        scratch_types=[
            pltpu.VMEM((_B * _D,), jnp.float32),
            pltpu.VMEM((_RB * _D + _D,), jnp.float32),
            pltpu.VMEM((_RB * _D + _D,), jnp.float32),
            pltpu.VMEM((_NBORD * _RB + 16,), jnp.int32),
            pltpu.VMEM((_B,), jnp.float32),
            pltpu.VMEM((_B,), jnp.float32),
            pltpu.VMEM((_NSLOT * _D,), jnp.float32),
            pltpu.SemaphoreType.DMA,
            pltpu.SemaphoreType.DMA,
            pltpu.SemaphoreType.DMA,
            pltpu.SemaphoreType.DMA,
        ],
    )

    lstm0_call = pl.pallas_call(
        _lstm0_body,
        out_shape=(jax.ShapeDtypeStruct((_B, d), jnp.float32),
                   jax.ShapeDtypeStruct((_B, d), jnp.float32)),
    )

    h, c = lstm0_call(bias)
    q_star = None
    for _ in range(_STEPS):
        mpf, spf, vpf = sc_attn(x1, batch_i, h.reshape(-1))
        q_star, h, c = fused_call(h, c, mpf.reshape(_NW, _B),
                                  spf.reshape(_NW, _B),
                                  vpf.reshape(_NW, _B, d),
                                  W_ih, W_hh, bias)
    return q_star


# speedup vs baseline: 1.0395x; 1.0008x over previous
import jax
import jax.numpy as jnp
from jax import lax
from jax.experimental import pallas as pl
from jax.experimental.pallas import tpu as pltpu
from jax.experimental.pallas import tpu_sc as plsc

_STEPS = 3
_B = 512
_D = 128
_NEG = -1e30

_RB = 160
_NBLK = 625
_NW = 32
_NBORD = (_NBLK + _NW - 1) // _NW
_NSLOT = 4


def _sc_attn_body(x_hbm, b_hbm, q_hbm, mp_hbm, sp_hbm, vp_hbm,
                  q_loc, xb0, xb1, bb_all, m_loc, s_loc, stage,
                  sem, semb, semx0, semx1):
    nc = 2
    wid = lax.axis_index("s") * nc + lax.axis_index("c")

    pltpu.sync_copy(q_hbm, q_loc)

    neg16 = jnp.full((16,), _NEG, jnp.float32)
    zero16 = jnp.zeros((16,), jnp.float32)
    for i in range(_B // 16):
        m_loc[pl.ds(i * 16, 16)] = neg16
        s_loc[pl.ds(i * 16, 16)] = zero16

    def flush(cur_b, cur_m, cur_s, acc, fc):
        slot = lax.rem(fc, _NSLOT)

        @pl.when(fc >= _NSLOT)
        def _():
            pltpu.make_async_copy(
                vp_hbm.at[pl.ds(0, _D)],
                stage.at[pl.ds(slot * _D, _D)], sem).wait()

        for j in range(8):
            stage[pl.ds(slot * _D + 16 * j, 16)] = acc[j]
        pltpu.make_async_copy(
            stage.at[pl.ds(slot * _D, _D)],
            vp_hbm.at[pl.ds((wid * _B + cur_b) * _D, _D)], sem).start()
        base = (cur_b // 16) * 16
        lmask = lax.iota(jnp.int32, 16) == (cur_b - base)
        m_loc[pl.ds(base, 16)] = jnp.where(lmask, cur_m,
                                           m_loc[pl.ds(base, 16)])
        s_loc[pl.ds(base, 16)] = jnp.where(lmask, cur_s,
                                           s_loc[pl.ds(base, 16)])

    def make_row_body(xref, boff, vf16):
        def row_body(r, carry):
            cur_b, cur_m, cur_s, acc, fc = carry
            b = bb_all[pl.ds(boff + r, 16)][0]
            xv = []
            facc = zero16
            for j in range(8):
                xj = xref[pl.ds(r * _D + 16 * j, 16)]
                qj = q_loc[pl.ds(b * _D + 16 * j, 16)]
                xv.append(xj)
                facc = facc + xj * qj
            iota16 = lax.iota(jnp.int32, 16)
            e_v = facc
            for k in (1, 2, 4, 8):
                e_v = e_v + jnp.take(e_v, jnp.bitwise_xor(iota16, k))

            changed = b != cur_b
            do_flush = changed & (cur_b >= 0) & (cur_s[0] > 0.0)

            @pl.when(do_flush)
            def _():
                flush(cur_b, cur_m, cur_s, acc, fc)

            fc = jnp.where(do_flush, fc + 1, fc)
            keep = lax.broadcast_in_dim(
                jnp.where(changed, 0.0, 1.0), (16,), ())
            m_prev = cur_m * keep + neg16 * (1.0 - keep)
            s_prev = cur_s * keep
            m_new = jnp.maximum(m_prev, e_v)
            f_old = jnp.exp(m_prev - m_new)
            wgt = jnp.exp(e_v - m_new) * vf16
            s_new = s_prev * f_old + wgt
            acc_new = tuple(
                acc[j] * (keep * f_old) + wgt * xv[j] for j in range(8))
            return (b, m_new, s_new, acc_new, fc)
        return row_body

    def _blk(ib):
        blk_i = wid + ib * _NW
        return jnp.where(blk_i < _NBLK, blk_i, wid)

    def bpre(ib, z):
        pltpu.make_async_copy(b_hbm.at[pl.ds(_blk(ib) * _RB, _RB)],
                              bb_all.at[pl.ds(ib * _RB, _RB)], semb).start()
        return z
    lax.fori_loop(0, _NBORD, bpre, 0)

    def bdrain(ib, z):
        pltpu.make_async_copy(b_hbm.at[pl.ds(0, _RB)],
                              bb_all.at[pl.ds(0, _RB)], semb).wait()
        return z
    lax.fori_loop(0, _NBORD, bdrain, 0)

    def xdesc(ib, bref, bsem):
        return pltpu.make_async_copy(
            x_hbm.at[pl.ds(_blk(ib) * _RB * _D, _RB * _D)],
            bref.at[pl.ds(0, _RB * _D)], bsem)

    xdesc(0, xb0, semx0).start()
    xdesc(1, xb1, semx1).start()

    def blk_pair(i, carry):
        for par, bref, bsem in ((0, xb0, semx0), (1, xb1, semx1)):
            ib = i * 2 + par
            xdesc(ib, bref, bsem).wait()
            valid = (wid + ib * _NW) < _NBLK
            vf16 = lax.broadcast_in_dim(
                jnp.where(valid, 1.0, 0.0), (16,), ())
            carry = lax.fori_loop(0, _RB,
                                  make_row_body(bref, ib * _RB, vf16), carry)

            @pl.when(ib + 2 < _NBORD)
            def _(ib=ib, bref=bref, bsem=bsem):
                xdesc(ib + 2, bref, bsem).start()
        return carry

    init = (jnp.int32(-1), neg16, zero16, (zero16,) * 8, jnp.int32(0))
    cur_b, cur_m, cur_s, acc, fc = lax.fori_loop(
        0, _NBORD // 2, blk_pair, init)

    last_real = (cur_b >= 0) & (cur_s[0] > 0.0)

    @pl.when(last_real)
    def _():
        flush(cur_b, cur_m, cur_s, acc, fc)
    fc = jnp.where(last_real, fc + 1, fc)

    def drain(i, _):
        pltpu.make_async_copy(
            vp_hbm.at[pl.ds(0, _D)], stage.at[pl.ds(0, _D)], sem).wait()
        return 0
    lax.fori_loop(0, jnp.minimum(fc, _NSLOT), drain, 0)

    pltpu.sync_copy(m_loc, mp_hbm.at[pl.ds(wid * _B, _B)])
    pltpu.sync_copy(s_loc, sp_hbm.at[pl.ds(wid * _B, _B)])


def _lstm0_body(bias_ref, h_out, c_out):
    d = h_out.shape[1]
    gates = jnp.broadcast_to(bias_ref[...], (_B, 4 * d))
    i = jax.nn.sigmoid(gates[:, :d])
    g = jnp.tanh(gates[:, 2 * d:3 * d])
    c_new = i * g
    h_out[...] = jax.nn.sigmoid(gates[:, 3 * d:]) * jnp.tanh(c_new)
    c_out[...] = c_new


def _fused_body(q_ref, c_ref, mp_ref, sp_ref, vp_ref,
                wih_ref, whh_ref, bias_ref, qs_out, h_out, c_out):
    d = q_ref.shape[1]
    r_iota = jax.lax.broadcasted_iota(jnp.int32, (_B, _B), 0)
    c_iota = jax.lax.broadcasted_iota(jnp.int32, (_B, _B), 1)
    ident = (r_iota == c_iota).astype(jnp.float32)
    mp_t = jax.lax.dot_general(ident, mp_ref[...], (((1,), (1,)), ((), ())),
                               preferred_element_type=jnp.float32)
    sp_t = jax.lax.dot_general(ident, sp_ref[...], (((1,), (1,)), ((), ())),
                               preferred_element_type=jnp.float32)
    alive = mp_t > (_NEG / 2)
    m = jnp.max(mp_t, axis=1, keepdims=True)
    coef = jnp.where(alive, jnp.exp(mp_t - m), 0.0)
    s = jnp.sum(coef * sp_t, axis=1, keepdims=True)
    v = jnp.zeros((_B, _D), jnp.float32)
    for w in range(_NW):
        cw = coef[:, w:w + 1]
        v = v + jnp.where(alive[:, w:w + 1], vp_ref[w] * cw, 0.0)
    r = v / (s + 1e-16)
    q_star = jnp.concatenate([q_ref[...], r], axis=1)
    qs_out[...] = q_star
    gates = (
        jax.lax.dot_general(q_star, wih_ref[...], (((1,), (1,)), ((), ())),
                            preferred_element_type=jnp.float32)
        + jax.lax.dot_general(q_ref[...], whh_ref[...],
                              (((1,), (1,)), ((), ())),
                              preferred_element_type=jnp.float32)
        + bias_ref[...]
    )
    i = jax.nn.sigmoid(gates[:, :d])
    f = jax.nn.sigmoid(gates[:, d:2 * d])
    g = jnp.tanh(gates[:, 2 * d:3 * d])
    o = jax.nn.sigmoid(gates[:, 3 * d:])
    c_new = f * c_ref[...] + i * g
    h_out[...] = o * jnp.tanh(c_new)
    c_out[...] = c_new


def kernel(x, batch, W_ih, W_hh, b_ih, b_hh):
    N, d = x.shape
    batch_i = batch.astype(jnp.int32)
    bias = (b_ih + b_hh).reshape(1, 4 * d).astype(jnp.float32)
    x1 = x.reshape(-1)

    fused_call = pl.pallas_call(
        _fused_body,
        out_shape=(jax.ShapeDtypeStruct((_B, 2 * d), jnp.float32),
                   jax.ShapeDtypeStruct((_B, d), jnp.float32),
                   jax.ShapeDtypeStruct((_B, d), jnp.float32)),
    )

    mesh = plsc.VectorSubcoreMesh(core_axis_name="c", subcore_axis_name="s")
    sc_attn = pl.kernel(
        _sc_attn_body,
        out_type=(jax.ShapeDtypeStruct((_NW * _B,), jnp.float32),
                  jax.ShapeDtypeStruct((_NW * _B,), jnp.float32),
                  jax.ShapeDtypeStruct((_NW * _B * _D,), jnp.float32)),
        mesh=mesh,
        scratch_types=[
            pltpu.VMEM((_B * _D,), jnp.float32),
            pltpu.VMEM((_RB * _D + _D,), jnp.float32),
            pltpu.VMEM((_RB * _D + _D,), jnp.float32),
            pltpu.VMEM((_NBORD * _RB + 16,), jnp.int32),
            pltpu.VMEM((_B,), jnp.float32),
            pltpu.VMEM((_B,), jnp.float32),
            pltpu.VMEM((_NSLOT * _D,), jnp.float32),
            pltpu.SemaphoreType.DMA,
            pltpu.SemaphoreType.DMA,
            pltpu.SemaphoreType.DMA,
            pltpu.SemaphoreType.DMA,
        ],
    )

    lstm0_call = pl.pallas_call(
        _lstm0_body,
        out_shape=(jax.ShapeDtypeStruct((_B, d), jnp.float32),
                   jax.ShapeDtypeStruct((_B, d), jnp.float32)),
    )

    h, c = lstm0_call(bias)
    q_star = None
    for _ in range(_STEPS):
        mpf, spf, vpf = sc_attn(x1, batch_i, h.reshape(-1))
        q_star, h, c = fused_call(h, c, mpf.reshape(_NW, _B),
                                  spf.reshape(_NW, _B),
                                  vpf.reshape(_NW, _B, d),
                                  W_ih, W_hh, bias)
    return q_star
